# Initial kernel scaffold; baseline (speedup 1.0000x reference)
import jax
import jax.numpy as jnp
from jax.experimental import pallas as pl


def kernel(x, edge_attr, pe, edge_index, batch, node_w, edge_w, pe_w, Wq, Wk, Wv, We, Wo, Woe, W1, b1, W2, b2, Wm1, bm1, Wm2, bm2, Wl1, bl1, Wl2, bl2):
    raise NotImplementedError("write your pallas kernel here")


# speedup vs baseline: 1.0002x; 1.0002x over previous
import functools

import jax
import jax.numpy as jnp
import numpy as np
from jax.experimental import pallas as pl
from jax.experimental.pallas import tpu as pltpu

N = 10000
E = 160000
D = 128
H = 8
G = 64


def _bn(x, eps=1e-5):
    m = jnp.mean(x, axis=0)
    v = jnp.var(x, axis=0)
    return (x - m) / jnp.sqrt(v + eps)


def _readout_kernel(h_ref, batch_ref, wm1_ref, bm1_ref, wm2_ref, bm2_ref,
                    wl1_ref, bl1_ref, wl2_ref, bl2_ref, mu_ref, std_ref):
    h = h_ref[...]
    batch = batch_ref[...]
    seg = jax.lax.broadcasted_iota(jnp.int32, (N, G), 1)
    onehot = (batch == seg).astype(jnp.float32)
    g = jax.lax.dot_general(onehot, h, (((0,), (0,)), ((), ())),
                            preferred_element_type=jnp.float32,
                            precision=jax.lax.Precision.HIGHEST)
    dot = functools.partial(jnp.dot, precision=jax.lax.Precision.HIGHEST)
    mu = dot(jax.nn.relu(dot(g, wm1_ref[...]) + bm1_ref[...]), wm2_ref[...]) + bm2_ref[...]
    log_var = dot(jax.nn.relu(dot(g, wl1_ref[...]) + bl1_ref[...]), wl2_ref[...]) + bl2_ref[...]
    mu_ref[...] = mu
    std_ref[...] = jnp.exp(0.5 * log_var)


def _readout(h, batch, Wm1, bm1, Wm2, bm2, Wl1, bl1, Wl2, bl2):
    return pl.pallas_call(
        _readout_kernel,
        out_shape=(jax.ShapeDtypeStruct((G, 1), jnp.float32),
                   jax.ShapeDtypeStruct((G, 1), jnp.float32)),
    )(h, batch.reshape(N, 1), Wm1, bm1.reshape(1, D), Wm2, bm2.reshape(1, 1),
      Wl1, bl1.reshape(1, D), Wl2, bl2.reshape(1, 1))


def kernel(x, edge_attr, pe, edge_index, batch, node_w, edge_w, pe_w, Wq, Wk, Wv, We, Wo, Woe, W1, b1, W2, b2, Wm1, bm1, Wm2, bm2, Wl1, bl1, Wl2, bl2):
    n = x.shape[0]
    src = edge_index[0]
    dst = edge_index[1]
    h = x @ node_w + pe @ pe_w
    e = edge_attr @ edge_w
    nlayers = Wq.shape[0]
    dh = D // H
    scale = float(np.sqrt(dh))
    for l in range(nlayers):
        q = (h @ Wq[l]).reshape(n, H, dh)
        k = (h @ Wk[l]).reshape(n, H, dh)
        v = (h @ Wv[l]).reshape(n, H, dh)
        ee = (e @ We[l]).reshape(-1, H, dh)
        qk = q[dst] * k[src] * ee
        score = qk.sum(-1) / scale
        smax = jax.ops.segment_max(score, dst, num_segments=n)
        smax = jnp.where(jnp.isfinite(smax), smax, 0.0)
        ex = jnp.exp(score - smax[dst])
        ssum = jax.ops.segment_sum(ex, dst, num_segments=n)
        attn = ex / (ssum[dst] + 1e-16)
        agg = jax.ops.segment_sum(attn[..., None] * v[src], dst, num_segments=n)
        h1 = _bn(h + agg.reshape(n, D) @ Wo[l])
        ff = jax.nn.relu(h1 @ W1[l] + b1[l]) @ W2[l] + b2[l]
        h = _bn(h1 + ff)
        e = _bn(e + qk.reshape(qk.shape[0], D) @ Woe[l])
    return _readout(h, batch, Wm1, bm1, Wm2, bm2, Wl1, bl1, Wl2, bl2)


# speedup vs baseline: 1.1069x; 1.1067x over previous
import jax
import jax.numpy as jnp
import numpy as np
from jax import lax
from jax.experimental import pallas as pl
from jax.experimental.pallas import tpu as pltpu
from jax.experimental.pallas import tpu_sc as plsc

N = 10000
E = 160000
EP = 163840
D = 128
H = 8
DH = 16
G = 64
C = 128
NCH = EP // (32 * C)


def _sc_gather_body(q_hbm, k_hbm, v_hbm, ee_hbm, dst_hbm, src_hbm,
                    qk_hbm, vs_hbm,
                    dstc, srcc, eev, qdv, ksv, vsv, sem):
    cid = lax.axis_index("c")
    sid = lax.axis_index("s")
    wid = sid * 2 + cid
    base_w = wid * (NCH * C)
    nchunks = jnp.minimum(NCH, (E - base_w) // C)

    def chunk(j, carry):
        base = base_w + j * C
        pltpu.sync_copy(dst_hbm.at[pl.ds(base, C)], dstc)
        pltpu.sync_copy(src_hbm.at[pl.ds(base, C)], srcc)
        pltpu.sync_copy(ee_hbm.at[pl.ds(base, C)], eev)
        c1 = pltpu.async_copy(q_hbm.at[dstc], qdv, sem)
        c2 = pltpu.async_copy(k_hbm.at[srcc], ksv, sem)
        c3 = pltpu.async_copy(v_hbm.at[srcc], vsv, sem)
        c1.wait()
        c2.wait()
        c3.wait()

        def edge(i, carry2):
            for hh in range(H):
                sl = pl.ds(hh * DH, DH)
                eev[i, sl] = qdv[i, sl] * ksv[i, sl] * eev[i, sl]
            return carry2

        lax.fori_loop(0, C, edge, 0)
        pltpu.sync_copy(eev, qk_hbm.at[pl.ds(base, C)])
        pltpu.sync_copy(vsv, vs_hbm.at[pl.ds(base, C)])
        return carry

    lax.fori_loop(0, nchunks, chunk, 0)


_sc_gather = pl.kernel(
    _sc_gather_body,
    out_type=(jax.ShapeDtypeStruct((EP, D), jnp.float32),
              jax.ShapeDtypeStruct((EP, D), jnp.float32)),
    mesh=plsc.VectorSubcoreMesh(core_axis_name="c", subcore_axis_name="s"),
    scratch_types=[
        pltpu.VMEM((C,), jnp.int32),
        pltpu.VMEM((C,), jnp.int32),
        pltpu.VMEM((C, D), jnp.float32),
        pltpu.VMEM((C, D), jnp.float32),
        pltpu.VMEM((C, D), jnp.float32),
        pltpu.VMEM((C, D), jnp.float32),
        pltpu.SemaphoreType.DMA,
    ],
)


def _bn(x, eps=1e-5):
    m = jnp.mean(x, axis=0)
    v = jnp.var(x, axis=0)
    return (x - m) / jnp.sqrt(v + eps)


def kernel(x, edge_attr, pe, edge_index, batch, node_w, edge_w, pe_w, Wq, Wk,
           Wv, We, Wo, Woe, W1, b1, W2, b2, Wm1, bm1, Wm2, bm2, Wl1, bl1,
           Wl2, bl2):
    n = x.shape[0]
    epad = EP - E
    srcp = jnp.concatenate([edge_index[0], jnp.zeros((epad,), jnp.int32)])
    dstp = jnp.concatenate([edge_index[1], jnp.zeros((epad,), jnp.int32)])
    dst = edge_index[1]
    h = x @ node_w + pe @ pe_w
    e = edge_attr @ edge_w
    nlayers = Wq.shape[0]
    dh = D // H
    scale = float(np.sqrt(dh))
    for l in range(nlayers):
        q = h @ Wq[l]
        k = h @ Wk[l]
        v = h @ Wv[l]
        ee = e @ We[l]
        eep = jnp.concatenate([ee, jnp.zeros((epad, D), jnp.float32)])
        qk2d, vs2d = _sc_gather(q, k, v, eep, dstp, srcp)
        qk = qk2d[:E].reshape(E, H, dh)
        vs = vs2d[:E].reshape(E, H, dh)
        score = qk.sum(-1) / scale
        smax = jax.ops.segment_max(score, dst, num_segments=n)
        smax = jnp.where(jnp.isfinite(smax), smax, 0.0)
        ex = jnp.exp(score - smax[dst])
        ssum = jax.ops.segment_sum(ex, dst, num_segments=n)
        attn = ex / (ssum[dst] + 1e-16)
        agg = jax.ops.segment_sum(attn[..., None] * vs, dst, num_segments=n)
        h1 = _bn(h + agg.reshape(n, D) @ Wo[l])
        ff = jax.nn.relu(h1 @ W1[l] + b1[l]) @ W2[l] + b2[l]
        h = _bn(h1 + ff)
        e = _bn(e + qk.reshape(E, D) @ Woe[l])
    g = jax.ops.segment_sum(h, batch, num_segments=G)
    mu = jax.nn.relu(g @ Wm1 + bm1) @ Wm2 + bm2
    log_var = jax.nn.relu(g @ Wl1 + bl1) @ Wl2 + bl2
    std = jnp.exp(0.5 * log_var)
    return (mu, std)
